# baseline (speedup 1.0000x reference)
import jax
import jax.numpy as jnp
from jax import lax
from jax.experimental import pallas as pl
from jax.experimental.pallas import tpu as pltpu
from jax.experimental.pallas import tpu_sc as plsc

N_NODES = 10000
D_FEAT = 128
N_EDGES = 320000

NC = 2
NS = 16
LANES = 128

N_PAD = 10240
ROWS_PER_TILE = N_PAD // NS
G_PER_TILE = 80
CHUNK = 16
G_TOTAL = NC * NS * G_PER_TILE
E_PAD = G_TOTAL * LANES


def _sc_segment_sum(with_cnt: bool):
    out_type = [jax.ShapeDtypeStruct((NC, N_PAD, D_FEAT), jnp.float32)]
    scratch = [
        pltpu.VMEM_SHARED((N_PAD, D_FEAT), jnp.float32),
        pltpu.VMEM((CHUNK, LANES), jnp.int32),
        pltpu.VMEM((CHUNK, LANES), jnp.int32),
        pltpu.VMEM((LANES, D_FEAT), jnp.float32),
        pltpu.VMEM((LANES, D_FEAT), jnp.float32),
        pltpu.SemaphoreType.DMA,
        pltpu.SemaphoreType.DMA,
        pltpu.SemaphoreType.DMA,
        pltpu.SemaphoreType.DMA,
    ]
    if with_cnt:
        out_type.append(jax.ShapeDtypeStruct((NC, N_PAD), jnp.float32))
        scratch.append(pltpu.VMEM_SHARED((N_PAD,), jnp.float32))
        scratch.append(pltpu.VMEM((LANES,), jnp.float32))
        scratch.append(pltpu.VMEM((LANES,), jnp.float32))

    mesh = plsc.VectorSubcoreMesh(core_axis_name="c", subcore_axis_name="s")

    def body(table_hbm, srcg_hbm, dstg_hbm, *refs):
        if with_cnt:
            (feat_out, cnt_out, agg_sh, src_v, dst_v, rows0_v, rows1_v,
             sem0, sem1, ssem0, ssem1, cnt_sh, ones_v, zflat_v) = refs
        else:
            (feat_out, agg_sh, src_v, dst_v, rows0_v, rows1_v,
             sem0, sem1, ssem0, ssem1) = refs
        c = lax.axis_index("c")
        s = lax.axis_index("s")
        wid = s * NC + c

        def fill_z(t, _):
            rows0_v[t // 8, pl.ds((t % 8) * 16, 16)] = jnp.zeros((16,), jnp.float32)
            return _
        lax.fori_loop(0, LANES * 8, fill_z, None)
        if with_cnt:
            def fill_o(t, _):
                ones_v[pl.ds(t * 16, 16)] = jnp.ones((16,), jnp.float32)
                zflat_v[pl.ds(t * 16, 16)] = jnp.zeros((16,), jnp.float32)
                return _
            lax.fori_loop(0, LANES // 16, fill_o, None)

        row0 = s * ROWS_PER_TILE
        def zero_blk(k, _):
            pltpu.sync_copy(rows0_v, agg_sh.at[pl.ds(row0 + k * LANES, LANES), :])
            return _
        lax.fori_loop(0, ROWS_PER_TILE // LANES, zero_blk, None)
        if with_cnt:
            def zero_cnt(k, _):
                pltpu.sync_copy(zflat_v, cnt_sh.at[pl.ds(row0 + k * LANES, LANES)])
                return _
            lax.fori_loop(0, ROWS_PER_TILE // LANES, zero_cnt, None)
        plsc.subcore_barrier()

        g0 = wid * G_PER_TILE

        def do_chunk(ci, _):
            pltpu.sync_copy(srcg_hbm.at[pl.ds(g0 + ci * CHUNK, CHUNK), :], src_v)
            pltpu.sync_copy(dstg_hbm.at[pl.ds(g0 + ci * CHUNK, CHUNK), :], dst_v)
            pltpu.async_copy(table_hbm.at[src_v.at[0]], rows0_v, sem0)
            pltpu.async_copy(table_hbm.at[src_v.at[1]], rows1_v, sem1)

            def pair(j2, _):
                e = 2 * j2
                pltpu.make_async_copy(table_hbm.at[src_v.at[e]],
                                      rows0_v, sem0).wait()
                pltpu.async_copy(rows0_v, agg_sh.at[dst_v.at[e]], ssem0,
                                 add=True)
                if with_cnt:
                    pltpu.sync_copy(ones_v, cnt_sh.at[dst_v.at[e]], add=True)

                pltpu.make_async_copy(table_hbm.at[src_v.at[e + 1]],
                                      rows1_v, sem1).wait()
                pltpu.async_copy(rows1_v, agg_sh.at[dst_v.at[e + 1]], ssem1,
                                 add=True)
                if with_cnt:
                    pltpu.sync_copy(ones_v, cnt_sh.at[dst_v.at[e + 1]], add=True)

                pltpu.make_async_copy(rows0_v, agg_sh.at[dst_v.at[e]],
                                      ssem0).wait()

                @pl.when(e + 2 < CHUNK)
                def _start0():
                    pltpu.async_copy(table_hbm.at[src_v.at[e + 2]],
                                     rows0_v, sem0)

                pltpu.make_async_copy(rows1_v, agg_sh.at[dst_v.at[e + 1]],
                                      ssem1).wait()

                @pl.when(e + 3 < CHUNK)
                def _start1():
                    pltpu.async_copy(table_hbm.at[src_v.at[e + 3]],
                                     rows1_v, sem1)
                return _

            lax.fori_loop(0, CHUNK // 2, pair, None)
            return _

        lax.fori_loop(0, G_PER_TILE // CHUNK, do_chunk, None)
        plsc.subcore_barrier()

        pltpu.sync_copy(agg_sh.at[pl.ds(row0, ROWS_PER_TILE), :],
                        feat_out.at[c, pl.ds(row0, ROWS_PER_TILE), :])
        if with_cnt:
            pltpu.sync_copy(cnt_sh.at[pl.ds(row0, ROWS_PER_TILE)],
                            cnt_out.at[c, pl.ds(row0, ROWS_PER_TILE)])

    return pl.kernel(body, out_type=tuple(out_type), mesh=mesh,
                     scratch_types=tuple(scratch))


def _tc_layer(feat_p, cnt_r, x, Wl, Wr, b, relu: bool):
    blk = 128
    grid = (pl.cdiv(N_NODES, blk),)

    def body(feat_ref, cnt_ref, x_ref, wl_ref, wr_ref, b_ref, o_ref):
        i = pl.program_id(0)
        ssum = feat_ref[0] + feat_ref[1]
        cnt = cnt_ref[0, i, :] + cnt_ref[1, i, :]
        inv = (1.0 / jnp.maximum(cnt, 1.0))[None, :]
        ii = lax.broadcasted_iota(jnp.int32, (blk, blk), 0)
        jj = lax.broadcasted_iota(jnp.int32, (blk, blk), 1)
        dmat = jnp.where(ii == jj, inv, 0.0)
        mean = jnp.dot(dmat, ssum, preferred_element_type=jnp.float32)
        acc = jnp.dot(mean, wl_ref[...], preferred_element_type=jnp.float32)
        acc += jnp.dot(x_ref[...], wr_ref[...], preferred_element_type=jnp.float32)
        acc += b_ref[...]
        o_ref[...] = jnp.maximum(acc, 0.0) if relu else acc

    return pl.pallas_call(
        body,
        grid=grid,
        in_specs=[
            pl.BlockSpec((NC, blk, D_FEAT), lambda i: (0, i, 0)),
            pl.BlockSpec((NC, N_PAD // LANES, LANES), lambda i: (0, 0, 0)),
            pl.BlockSpec((blk, D_FEAT), lambda i: (i, 0)),
            pl.BlockSpec((D_FEAT, D_FEAT), lambda i: (0, 0)),
            pl.BlockSpec((D_FEAT, D_FEAT), lambda i: (0, 0)),
            pl.BlockSpec((1, D_FEAT), lambda i: (0, 0)),
        ],
        out_specs=pl.BlockSpec((blk, D_FEAT), lambda i: (i, 0)),
        out_shape=jax.ShapeDtypeStruct((N_NODES, D_FEAT), jnp.float32),
    )(feat_p, cnt_r, x, Wl, Wr, b.reshape(1, D_FEAT))


def kernel(x, edge_index, W1l, b1, W1r, W2l, b2, W2r):
    src = edge_index[0].astype(jnp.int32)
    dst = edge_index[1].astype(jnp.int32)

    pad = E_PAD - N_EDGES
    ar = jnp.arange(pad, dtype=jnp.int32)
    src_p = jnp.concatenate([src, ar % N_NODES])
    dst_p = jnp.concatenate([dst, N_NODES + ar % (N_PAD - N_NODES)])
    srcg = src_p.reshape(G_TOTAL, LANES)
    dstg = dst_p.reshape(G_TOTAL, LANES)

    feat1_p, cnt_p = _sc_segment_sum(True)(x, srcg, dstg)
    cnt_r = cnt_p.reshape(NC, N_PAD // LANES, LANES)
    h = _tc_layer(feat1_p, cnt_r, x, W1l, W1r, b1, relu=True)
    (feat2_p,) = _sc_segment_sum(False)(h, srcg, dstg)
    out = _tc_layer(feat2_p, cnt_r, h, W2l, W2r, b2, relu=False)
    return out


# speedup vs baseline: 1.1755x; 1.1755x over previous
import jax
import jax.numpy as jnp
from jax import lax
from jax.experimental import pallas as pl
from jax.experimental.pallas import tpu as pltpu
from jax.experimental.pallas import tpu_sc as plsc

N_NODES = 10000
D_FEAT = 128
N_EDGES = 320000

NC = 2
NS = 16
LANES = 128

N_PAD = 10240
ROWS_PER_TILE = N_PAD // NS
G_PER_TILE = 80
CHUNK = 16
G_TOTAL = NC * NS * G_PER_TILE
E_PAD = G_TOTAL * LANES


def _sc_segment_sum(with_cnt: bool):
    out_type = [jax.ShapeDtypeStruct((NC, N_PAD, D_FEAT), jnp.float32)]
    scratch = [
        pltpu.VMEM_SHARED((N_PAD, D_FEAT), jnp.float32),
        pltpu.VMEM((CHUNK, LANES), jnp.int32),
        pltpu.VMEM((CHUNK, LANES), jnp.int32),
        pltpu.VMEM((LANES, D_FEAT), jnp.float32),
        pltpu.VMEM((LANES, D_FEAT), jnp.float32),
        pltpu.SemaphoreType.DMA,
        pltpu.SemaphoreType.DMA,
    ]
    if with_cnt:
        out_type.append(jax.ShapeDtypeStruct((NC, N_PAD), jnp.float32))
        scratch.append(pltpu.VMEM_SHARED((N_PAD,), jnp.float32))
        scratch.append(pltpu.VMEM((LANES,), jnp.float32))
        scratch.append(pltpu.VMEM((LANES,), jnp.float32))

    mesh = plsc.VectorSubcoreMesh(core_axis_name="c", subcore_axis_name="s")

    def body(table_hbm, srcg_hbm, dstg_hbm, *refs):
        if with_cnt:
            (feat_out, cnt_out, agg_sh, src_v, dst_v, rows0_v, rows1_v,
             sem0, sem1, cnt_sh, ones_v, zflat_v) = refs
        else:
            (feat_out, agg_sh, src_v, dst_v, rows0_v, rows1_v,
             sem0, sem1) = refs
        c = lax.axis_index("c")
        s = lax.axis_index("s")
        wid = s * NC + c

        def fill_z(t, _):
            rows0_v[t // 8, pl.ds((t % 8) * 16, 16)] = jnp.zeros((16,), jnp.float32)
            return _
        lax.fori_loop(0, LANES * 8, fill_z, None)
        if with_cnt:
            def fill_o(t, _):
                ones_v[pl.ds(t * 16, 16)] = jnp.ones((16,), jnp.float32)
                zflat_v[pl.ds(t * 16, 16)] = jnp.zeros((16,), jnp.float32)
                return _
            lax.fori_loop(0, LANES // 16, fill_o, None)

        row0 = s * ROWS_PER_TILE
        def zero_blk(k, _):
            pltpu.sync_copy(rows0_v, agg_sh.at[pl.ds(row0 + k * LANES, LANES), :])
            return _
        lax.fori_loop(0, ROWS_PER_TILE // LANES, zero_blk, None)
        if with_cnt:
            def zero_cnt(k, _):
                pltpu.sync_copy(zflat_v, cnt_sh.at[pl.ds(row0 + k * LANES, LANES)])
                return _
            lax.fori_loop(0, ROWS_PER_TILE // LANES, zero_cnt, None)
        plsc.subcore_barrier()

        g0 = wid * G_PER_TILE

        def do_chunk(ci, _):
            pltpu.sync_copy(srcg_hbm.at[pl.ds(g0 + ci * CHUNK, CHUNK), :], src_v)
            pltpu.sync_copy(dstg_hbm.at[pl.ds(g0 + ci * CHUNK, CHUNK), :], dst_v)
            pltpu.async_copy(table_hbm.at[src_v.at[0]], rows0_v, sem0)
            pltpu.async_copy(table_hbm.at[src_v.at[1]], rows1_v, sem1)

            def pair(j2, _):
                e = 2 * j2
                pltpu.make_async_copy(table_hbm.at[src_v.at[e]],
                                      rows0_v, sem0).wait()
                pltpu.sync_copy(rows0_v, agg_sh.at[dst_v.at[e]], add=True)
                if with_cnt:
                    pltpu.sync_copy(ones_v, cnt_sh.at[dst_v.at[e]], add=True)

                @pl.when(e + 2 < CHUNK)
                def _start0():
                    pltpu.async_copy(table_hbm.at[src_v.at[e + 2]],
                                     rows0_v, sem0)

                pltpu.make_async_copy(table_hbm.at[src_v.at[e + 1]],
                                      rows1_v, sem1).wait()
                pltpu.sync_copy(rows1_v, agg_sh.at[dst_v.at[e + 1]], add=True)
                if with_cnt:
                    pltpu.sync_copy(ones_v, cnt_sh.at[dst_v.at[e + 1]], add=True)

                @pl.when(e + 3 < CHUNK)
                def _start1():
                    pltpu.async_copy(table_hbm.at[src_v.at[e + 3]],
                                     rows1_v, sem1)
                return _

            lax.fori_loop(0, CHUNK // 2, pair, None)
            return _

        lax.fori_loop(0, G_PER_TILE // CHUNK, do_chunk, None)
        plsc.subcore_barrier()

        pltpu.sync_copy(agg_sh.at[pl.ds(row0, ROWS_PER_TILE), :],
                        feat_out.at[c, pl.ds(row0, ROWS_PER_TILE), :])
        if with_cnt:
            pltpu.sync_copy(cnt_sh.at[pl.ds(row0, ROWS_PER_TILE)],
                            cnt_out.at[c, pl.ds(row0, ROWS_PER_TILE)])

    return pl.kernel(body, out_type=tuple(out_type), mesh=mesh,
                     scratch_types=tuple(scratch))


def _tc_layer(feat_p, cnt_r, x, Wl, Wr, b, relu: bool):
    blk = 128
    grid = (pl.cdiv(N_NODES, blk),)

    def body(feat_ref, cnt_ref, x_ref, wl_ref, wr_ref, b_ref, o_ref):
        i = pl.program_id(0)
        ssum = feat_ref[0] + feat_ref[1]
        cnt = cnt_ref[0, i, :] + cnt_ref[1, i, :]
        inv = (1.0 / jnp.maximum(cnt, 1.0))[None, :]
        ii = lax.broadcasted_iota(jnp.int32, (blk, blk), 0)
        jj = lax.broadcasted_iota(jnp.int32, (blk, blk), 1)
        dmat = jnp.where(ii == jj, inv, 0.0)
        mean = jnp.dot(dmat, ssum, preferred_element_type=jnp.float32)
        acc = jnp.dot(mean, wl_ref[...], preferred_element_type=jnp.float32)
        acc += jnp.dot(x_ref[...], wr_ref[...], preferred_element_type=jnp.float32)
        acc += b_ref[...]
        o_ref[...] = jnp.maximum(acc, 0.0) if relu else acc

    return pl.pallas_call(
        body,
        grid=grid,
        in_specs=[
            pl.BlockSpec((NC, blk, D_FEAT), lambda i: (0, i, 0)),
            pl.BlockSpec((NC, N_PAD // LANES, LANES), lambda i: (0, 0, 0)),
            pl.BlockSpec((blk, D_FEAT), lambda i: (i, 0)),
            pl.BlockSpec((D_FEAT, D_FEAT), lambda i: (0, 0)),
            pl.BlockSpec((D_FEAT, D_FEAT), lambda i: (0, 0)),
            pl.BlockSpec((1, D_FEAT), lambda i: (0, 0)),
        ],
        out_specs=pl.BlockSpec((blk, D_FEAT), lambda i: (i, 0)),
        out_shape=jax.ShapeDtypeStruct((N_NODES, D_FEAT), jnp.float32),
    )(feat_p, cnt_r, x, Wl, Wr, b.reshape(1, D_FEAT))


def kernel(x, edge_index, W1l, b1, W1r, W2l, b2, W2r):
    src = edge_index[0].astype(jnp.int32)
    dst = edge_index[1].astype(jnp.int32)

    pad = E_PAD - N_EDGES
    ar = jnp.arange(pad, dtype=jnp.int32)
    src_p = jnp.concatenate([src, ar % N_NODES])
    dst_p = jnp.concatenate([dst, N_NODES + ar % (N_PAD - N_NODES)])
    srcg = src_p.reshape(G_TOTAL, LANES)
    dstg = dst_p.reshape(G_TOTAL, LANES)

    feat1_p, cnt_p = _sc_segment_sum(True)(x, srcg, dstg)
    cnt_r = cnt_p.reshape(NC, N_PAD // LANES, LANES)
    h = _tc_layer(feat1_p, cnt_r, x, W1l, W1r, b1, relu=True)
    (feat2_p,) = _sc_segment_sum(False)(h, srcg, dstg)
    out = _tc_layer(feat2_p, cnt_r, h, W2l, W2r, b2, relu=False)
    return out


# speedup vs baseline: 1.4979x; 1.2743x over previous
import jax
import jax.numpy as jnp
from jax import lax
from jax.experimental import pallas as pl
from jax.experimental.pallas import tpu as pltpu
from jax.experimental.pallas import tpu_sc as plsc

N_NODES = 10000
D_FEAT = 128
N_EDGES = 320000

NC = 2
NS = 16
LANES = 128

N_PAD = 10240
ROWS_PER_TILE = N_PAD // NS
G_PER_TILE = 80
CHUNK = 16
G_TOTAL = NC * NS * G_PER_TILE
E_PAD = G_TOTAL * LANES


def _sc_segment_sum(with_cnt: bool):
    out_type = [jax.ShapeDtypeStruct((NC, N_PAD, D_FEAT), jnp.float32)]
    scratch = [
        pltpu.VMEM_SHARED((N_PAD, D_FEAT), jnp.float32),
        pltpu.VMEM((CHUNK, LANES), jnp.int32),
        pltpu.VMEM((CHUNK, LANES), jnp.int32),
        pltpu.VMEM((LANES, D_FEAT), jnp.float32),
        pltpu.VMEM((LANES, D_FEAT), jnp.float32),
        pltpu.SemaphoreType.DMA,
        pltpu.SemaphoreType.DMA,
    ]
    if with_cnt:
        out_type.append(jax.ShapeDtypeStruct((NC, N_PAD), jnp.float32))
        scratch.append(pltpu.VMEM_SHARED((N_PAD,), jnp.float32))
        scratch.append(pltpu.VMEM((LANES,), jnp.float32))
        scratch.append(pltpu.VMEM((LANES,), jnp.float32))

    mesh = plsc.VectorSubcoreMesh(core_axis_name="c", subcore_axis_name="s")

    def body(table_hbm, srcg_hbm, dstg_hbm, *refs):
        if with_cnt:
            (feat_out, cnt_out, agg_sh, src_v, dst_v, rows0_v, rows1_v,
             sem0, sem1, cnt_sh, ones_v, zflat_v) = refs
        else:
            (feat_out, agg_sh, src_v, dst_v, rows0_v, rows1_v,
             sem0, sem1) = refs
        c = lax.axis_index("c")
        s = lax.axis_index("s")
        wid = s * NC + c

        def fill_z(t, _):
            rows0_v[t // 8, pl.ds((t % 8) * 16, 16)] = jnp.zeros((16,), jnp.float32)
            return _
        lax.fori_loop(0, LANES * 8, fill_z, None)
        if with_cnt:
            def fill_o(t, _):
                ones_v[pl.ds(t * 16, 16)] = jnp.ones((16,), jnp.float32)
                zflat_v[pl.ds(t * 16, 16)] = jnp.zeros((16,), jnp.float32)
                return _
            lax.fori_loop(0, LANES // 16, fill_o, None)

        row0 = s * ROWS_PER_TILE
        def zero_blk(k, _):
            pltpu.sync_copy(rows0_v, agg_sh.at[pl.ds(row0 + k * LANES, LANES), :])
            return _
        lax.fori_loop(0, ROWS_PER_TILE // LANES, zero_blk, None)
        if with_cnt:
            def zero_cnt(k, _):
                pltpu.sync_copy(zflat_v, cnt_sh.at[pl.ds(row0 + k * LANES, LANES)])
                return _
            lax.fori_loop(0, ROWS_PER_TILE // LANES, zero_cnt, None)
        plsc.subcore_barrier()

        g0 = wid * G_PER_TILE

        def do_chunk(ci, _):
            pltpu.sync_copy(srcg_hbm.at[pl.ds(g0 + ci * CHUNK, CHUNK), :], src_v)
            pltpu.sync_copy(dstg_hbm.at[pl.ds(g0 + ci * CHUNK, CHUNK), :], dst_v)
            pltpu.async_copy(table_hbm.at[src_v.at[0]], rows0_v, sem0)
            pltpu.async_copy(table_hbm.at[src_v.at[1]], rows1_v, sem1)

            def pair(j2, _):
                e = 2 * j2
                pltpu.make_async_copy(table_hbm.at[src_v.at[e]],
                                      rows0_v, sem0).wait()
                pltpu.sync_copy(rows0_v, agg_sh.at[dst_v.at[e]], add=True)
                if with_cnt:
                    pltpu.sync_copy(ones_v, cnt_sh.at[dst_v.at[e]], add=True)

                @pl.when(e + 2 < CHUNK)
                def _start0():
                    pltpu.async_copy(table_hbm.at[src_v.at[e + 2]],
                                     rows0_v, sem0)

                pltpu.make_async_copy(table_hbm.at[src_v.at[e + 1]],
                                      rows1_v, sem1).wait()
                pltpu.sync_copy(rows1_v, agg_sh.at[dst_v.at[e + 1]], add=True)
                if with_cnt:
                    pltpu.sync_copy(ones_v, cnt_sh.at[dst_v.at[e + 1]], add=True)

                @pl.when(e + 3 < CHUNK)
                def _start1():
                    pltpu.async_copy(table_hbm.at[src_v.at[e + 3]],
                                     rows1_v, sem1)
                return _

            lax.fori_loop(0, CHUNK // 2, pair, None)
            return _

        lax.fori_loop(0, G_PER_TILE // CHUNK, do_chunk, None)
        plsc.subcore_barrier()

        pltpu.sync_copy(agg_sh.at[pl.ds(row0, ROWS_PER_TILE), :],
                        feat_out.at[c, pl.ds(row0, ROWS_PER_TILE), :])
        if with_cnt:
            pltpu.sync_copy(cnt_sh.at[pl.ds(row0, ROWS_PER_TILE)],
                            cnt_out.at[c, pl.ds(row0, ROWS_PER_TILE)])

    return pl.kernel(body, out_type=tuple(out_type), mesh=mesh,
                     scratch_types=tuple(scratch))


def _tc_layer(feat_p, cnt_r, x, Wl, Wr, b, relu: bool):
    blk = 1280
    grid = (pl.cdiv(N_NODES, blk),)
    sub = blk // LANES

    def body(feat_ref, cnt_ref, x_ref, wl_ref, wr_ref, b_ref, o_ref):
        i = pl.program_id(0)
        ssum = feat_ref[0] + feat_ref[1]
        ii = lax.broadcasted_iota(jnp.int32, (LANES, LANES), 0)
        jj = lax.broadcasted_iota(jnp.int32, (LANES, LANES), 1)
        eye = ii == jj
        parts = []
        for k in range(sub):
            cnt = cnt_ref[0, i * sub + k, :] + cnt_ref[1, i * sub + k, :]
            inv = (1.0 / jnp.maximum(cnt, 1.0))[None, :]
            dmat = jnp.where(eye, inv, 0.0)
            parts.append(jnp.dot(dmat, ssum[k * LANES:(k + 1) * LANES],
                                 preferred_element_type=jnp.float32))
        mean = jnp.concatenate(parts, axis=0)
        acc = jnp.dot(mean, wl_ref[...], preferred_element_type=jnp.float32)
        acc += jnp.dot(x_ref[...], wr_ref[...], preferred_element_type=jnp.float32)
        acc += b_ref[...]
        o_ref[...] = jnp.maximum(acc, 0.0) if relu else acc

    return pl.pallas_call(
        body,
        grid=grid,
        in_specs=[
            pl.BlockSpec((NC, blk, D_FEAT), lambda i: (0, i, 0)),
            pl.BlockSpec((NC, N_PAD // LANES, LANES), lambda i: (0, 0, 0)),
            pl.BlockSpec((blk, D_FEAT), lambda i: (i, 0)),
            pl.BlockSpec((D_FEAT, D_FEAT), lambda i: (0, 0)),
            pl.BlockSpec((D_FEAT, D_FEAT), lambda i: (0, 0)),
            pl.BlockSpec((1, D_FEAT), lambda i: (0, 0)),
        ],
        out_specs=pl.BlockSpec((blk, D_FEAT), lambda i: (i, 0)),
        out_shape=jax.ShapeDtypeStruct((N_NODES, D_FEAT), jnp.float32),
    )(feat_p, cnt_r, x, Wl, Wr, b.reshape(1, D_FEAT))


def kernel(x, edge_index, W1l, b1, W1r, W2l, b2, W2r):
    src = edge_index[0].astype(jnp.int32)
    dst = edge_index[1].astype(jnp.int32)

    pad = E_PAD - N_EDGES
    ar = jnp.arange(pad, dtype=jnp.int32)
    src_p = jnp.concatenate([src, ar % N_NODES])
    dst_p = jnp.concatenate([dst, N_NODES + ar % (N_PAD - N_NODES)])
    srcg = src_p.reshape(G_TOTAL, LANES)
    dstg = dst_p.reshape(G_TOTAL, LANES)

    feat1_p, cnt_p = _sc_segment_sum(True)(x, srcg, dstg)
    cnt_r = cnt_p.reshape(NC, N_PAD // LANES, LANES)
    h = _tc_layer(feat1_p, cnt_r, x, W1l, W1r, b1, relu=True)
    (feat2_p,) = _sc_segment_sum(False)(h, srcg, dstg)
    out = _tc_layer(feat2_p, cnt_r, h, W2l, W2r, b2, relu=False)
    return out


# speedup vs baseline: 1.6071x; 1.0729x over previous
import jax
import jax.numpy as jnp
from jax import lax
from jax.experimental import pallas as pl
from jax.experimental.pallas import tpu as pltpu
from jax.experimental.pallas import tpu_sc as plsc

N_NODES = 10000
D_FEAT = 128
N_EDGES = 320000

NC = 2
NS = 16
LANES = 128

N_PAD = 10240
ROWS_PER_TILE = N_PAD // NS
G_PER_TILE = 80
CHUNK = 40
G_TOTAL = NC * NS * G_PER_TILE
E_PAD = G_TOTAL * LANES


def _sc_segment_sum(with_cnt: bool):
    out_type = [jax.ShapeDtypeStruct((NC, N_PAD, D_FEAT), jnp.float32)]
    scratch = [
        pltpu.VMEM_SHARED((N_PAD, D_FEAT), jnp.float32),
        pltpu.VMEM((CHUNK, LANES), jnp.int32),
        pltpu.VMEM((CHUNK, LANES), jnp.int32),
        pltpu.VMEM((LANES, D_FEAT), jnp.float32),
        pltpu.VMEM((LANES, D_FEAT), jnp.float32),
        pltpu.SemaphoreType.DMA,
        pltpu.SemaphoreType.DMA,
    ]
    if with_cnt:
        out_type.append(jax.ShapeDtypeStruct((NC, N_PAD), jnp.float32))
        scratch.append(pltpu.VMEM_SHARED((N_PAD,), jnp.float32))
        scratch.append(pltpu.VMEM((LANES,), jnp.float32))
        scratch.append(pltpu.VMEM((LANES,), jnp.float32))

    mesh = plsc.VectorSubcoreMesh(core_axis_name="c", subcore_axis_name="s")

    def body(table_hbm, srcg_hbm, dstg_hbm, *refs):
        if with_cnt:
            (feat_out, cnt_out, agg_sh, src_v, dst_v, rows0_v, rows1_v,
             sem0, sem1, cnt_sh, ones_v, zflat_v) = refs
        else:
            (feat_out, agg_sh, src_v, dst_v, rows0_v, rows1_v,
             sem0, sem1) = refs
        c = lax.axis_index("c")
        s = lax.axis_index("s")
        wid = s * NC + c

        def fill_z(t, _):
            rows0_v[t // 8, pl.ds((t % 8) * 16, 16)] = jnp.zeros((16,), jnp.float32)
            return _
        lax.fori_loop(0, LANES * 8, fill_z, None)
        if with_cnt:
            def fill_o(t, _):
                ones_v[pl.ds(t * 16, 16)] = jnp.ones((16,), jnp.float32)
                zflat_v[pl.ds(t * 16, 16)] = jnp.zeros((16,), jnp.float32)
                return _
            lax.fori_loop(0, LANES // 16, fill_o, None)

        row0 = s * ROWS_PER_TILE
        def zero_blk(k, _):
            pltpu.sync_copy(rows0_v, agg_sh.at[pl.ds(row0 + k * LANES, LANES), :])
            return _
        lax.fori_loop(0, ROWS_PER_TILE // LANES, zero_blk, None)
        if with_cnt:
            def zero_cnt(k, _):
                pltpu.sync_copy(zflat_v, cnt_sh.at[pl.ds(row0 + k * LANES, LANES)])
                return _
            lax.fori_loop(0, ROWS_PER_TILE // LANES, zero_cnt, None)
        plsc.subcore_barrier()

        g0 = wid * G_PER_TILE

        def do_chunk(ci, _):
            pltpu.sync_copy(srcg_hbm.at[pl.ds(g0 + ci * CHUNK, CHUNK), :], src_v)
            pltpu.sync_copy(dstg_hbm.at[pl.ds(g0 + ci * CHUNK, CHUNK), :], dst_v)
            pltpu.async_copy(table_hbm.at[src_v.at[0]], rows0_v, sem0)
            pltpu.async_copy(table_hbm.at[src_v.at[1]], rows1_v, sem1)

            def pair(j2, _):
                e = 2 * j2
                pltpu.make_async_copy(table_hbm.at[src_v.at[e]],
                                      rows0_v, sem0).wait()
                pltpu.sync_copy(rows0_v, agg_sh.at[dst_v.at[e]], add=True)
                if with_cnt:
                    pltpu.sync_copy(ones_v, cnt_sh.at[dst_v.at[e]], add=True)

                @pl.when(e + 2 < CHUNK)
                def _start0():
                    pltpu.async_copy(table_hbm.at[src_v.at[e + 2]],
                                     rows0_v, sem0)

                pltpu.make_async_copy(table_hbm.at[src_v.at[e + 1]],
                                      rows1_v, sem1).wait()
                pltpu.sync_copy(rows1_v, agg_sh.at[dst_v.at[e + 1]], add=True)
                if with_cnt:
                    pltpu.sync_copy(ones_v, cnt_sh.at[dst_v.at[e + 1]], add=True)

                @pl.when(e + 3 < CHUNK)
                def _start1():
                    pltpu.async_copy(table_hbm.at[src_v.at[e + 3]],
                                     rows1_v, sem1)
                return _

            lax.fori_loop(0, CHUNK // 2, pair, None)
            return _

        lax.fori_loop(0, G_PER_TILE // CHUNK, do_chunk, None)
        plsc.subcore_barrier()

        pltpu.sync_copy(agg_sh.at[pl.ds(row0, ROWS_PER_TILE), :],
                        feat_out.at[c, pl.ds(row0, ROWS_PER_TILE), :])
        if with_cnt:
            pltpu.sync_copy(cnt_sh.at[pl.ds(row0, ROWS_PER_TILE)],
                            cnt_out.at[c, pl.ds(row0, ROWS_PER_TILE)])

    return pl.kernel(body, out_type=tuple(out_type), mesh=mesh,
                     scratch_types=tuple(scratch))


def _tc_layer(feat_p, cnt_r, x, Wl, Wr, b, relu: bool):
    blk = 2560
    grid = (pl.cdiv(N_NODES, blk),)
    sub = blk // LANES

    def body(feat_ref, cnt_ref, x_ref, wl_ref, wr_ref, b_ref, o_ref):
        i = pl.program_id(0)
        ssum = feat_ref[0] + feat_ref[1]
        ii = lax.broadcasted_iota(jnp.int32, (LANES, LANES), 0)
        jj = lax.broadcasted_iota(jnp.int32, (LANES, LANES), 1)
        eye = ii == jj
        parts = []
        for k in range(sub):
            cnt = cnt_ref[0, i * sub + k, :] + cnt_ref[1, i * sub + k, :]
            inv = (1.0 / jnp.maximum(cnt, 1.0))[None, :]
            dmat = jnp.where(eye, inv, 0.0)
            parts.append(jnp.dot(dmat, ssum[k * LANES:(k + 1) * LANES],
                                 preferred_element_type=jnp.float32))
        mean = jnp.concatenate(parts, axis=0)
        acc = jnp.dot(mean, wl_ref[...], preferred_element_type=jnp.float32)
        acc += jnp.dot(x_ref[...], wr_ref[...], preferred_element_type=jnp.float32)
        acc += b_ref[...]
        o_ref[...] = jnp.maximum(acc, 0.0) if relu else acc

    return pl.pallas_call(
        body,
        grid=grid,
        in_specs=[
            pl.BlockSpec((NC, blk, D_FEAT), lambda i: (0, i, 0)),
            pl.BlockSpec((NC, N_PAD // LANES, LANES), lambda i: (0, 0, 0)),
            pl.BlockSpec((blk, D_FEAT), lambda i: (i, 0)),
            pl.BlockSpec((D_FEAT, D_FEAT), lambda i: (0, 0)),
            pl.BlockSpec((D_FEAT, D_FEAT), lambda i: (0, 0)),
            pl.BlockSpec((1, D_FEAT), lambda i: (0, 0)),
        ],
        out_specs=pl.BlockSpec((blk, D_FEAT), lambda i: (i, 0)),
        out_shape=jax.ShapeDtypeStruct((N_NODES, D_FEAT), jnp.float32),
    )(feat_p, cnt_r, x, Wl, Wr, b.reshape(1, D_FEAT))


def kernel(x, edge_index, W1l, b1, W1r, W2l, b2, W2r):
    src = edge_index[0].astype(jnp.int32)
    dst = edge_index[1].astype(jnp.int32)

    pad = E_PAD - N_EDGES
    ar = jnp.arange(pad, dtype=jnp.int32)
    src_p = jnp.concatenate([src, ar % N_NODES])
    dst_p = jnp.concatenate([dst, N_NODES + ar % (N_PAD - N_NODES)])
    srcg = src_p.reshape(G_TOTAL, LANES)
    dstg = dst_p.reshape(G_TOTAL, LANES)

    feat1_p, cnt_p = _sc_segment_sum(True)(x, srcg, dstg)
    cnt_r = cnt_p.reshape(NC, N_PAD // LANES, LANES)
    h = _tc_layer(feat1_p, cnt_r, x, W1l, W1r, b1, relu=True)
    (feat2_p,) = _sc_segment_sum(False)(h, srcg, dstg)
    out = _tc_layer(feat2_p, cnt_r, h, W2l, W2r, b2, relu=False)
    return out


# speedup vs baseline: 1.6114x; 1.0027x over previous
import jax
import jax.numpy as jnp
from jax import lax
from jax.experimental import pallas as pl
from jax.experimental.pallas import tpu as pltpu
from jax.experimental.pallas import tpu_sc as plsc

N_NODES = 10000
D_FEAT = 128
N_EDGES = 320000

NC = 2
NS = 16
LANES = 128

N_PAD = 10240
ROWS_PER_TILE = N_PAD // NS
G_PER_TILE = 80
CHUNK = 40
G_TOTAL = NC * NS * G_PER_TILE
E_PAD = G_TOTAL * LANES


def _sc_segment_sum(with_cnt: bool):
    out_type = [jax.ShapeDtypeStruct((NC, N_PAD, D_FEAT), jnp.float32)]
    scratch = [
        pltpu.VMEM_SHARED((N_PAD, D_FEAT), jnp.float32),
        pltpu.VMEM((CHUNK, LANES), jnp.int32),
        pltpu.VMEM((CHUNK, LANES), jnp.int32),
        pltpu.VMEM((LANES, D_FEAT), jnp.float32),
        pltpu.VMEM((LANES, D_FEAT), jnp.float32),
        pltpu.SemaphoreType.DMA,
        pltpu.SemaphoreType.DMA,
    ]
    if with_cnt:
        out_type.append(jax.ShapeDtypeStruct((NC, N_PAD), jnp.float32))
        scratch.append(pltpu.VMEM_SHARED((N_PAD,), jnp.float32))
        scratch.append(pltpu.VMEM((LANES,), jnp.float32))
        scratch.append(pltpu.VMEM((LANES,), jnp.float32))

    mesh = plsc.VectorSubcoreMesh(core_axis_name="c", subcore_axis_name="s")

    def body(table_hbm, srcg_hbm, dstg_hbm, *refs):
        if with_cnt:
            (feat_out, cnt_out, agg_sh, src_v, dst_v, rows0_v, rows1_v,
             sem0, sem1, cnt_sh, ones_v, zflat_v) = refs
        else:
            (feat_out, agg_sh, src_v, dst_v, rows0_v, rows1_v,
             sem0, sem1) = refs
        c = lax.axis_index("c")
        s = lax.axis_index("s")
        wid = s * NC + c

        def fill_z(t, _):
            rows0_v[t // 8, pl.ds((t % 8) * 16, 16)] = jnp.zeros((16,), jnp.float32)
            return _
        lax.fori_loop(0, LANES * 8, fill_z, None)
        if with_cnt:
            def fill_o(t, _):
                ones_v[pl.ds(t * 16, 16)] = jnp.ones((16,), jnp.float32)
                zflat_v[pl.ds(t * 16, 16)] = jnp.zeros((16,), jnp.float32)
                return _
            lax.fori_loop(0, LANES // 16, fill_o, None)

        row0 = s * ROWS_PER_TILE
        def zero_blk(k, _):
            pltpu.sync_copy(rows0_v, agg_sh.at[pl.ds(row0 + k * LANES, LANES), :])
            return _
        lax.fori_loop(0, ROWS_PER_TILE // LANES, zero_blk, None)
        if with_cnt:
            def zero_cnt(k, _):
                pltpu.sync_copy(zflat_v, cnt_sh.at[pl.ds(row0 + k * LANES, LANES)])
                return _
            lax.fori_loop(0, ROWS_PER_TILE // LANES, zero_cnt, None)
        plsc.subcore_barrier()

        g0 = wid * G_PER_TILE

        def do_chunk(ci, _):
            pltpu.sync_copy(srcg_hbm.at[pl.ds(g0 + ci * CHUNK, CHUNK), :], src_v)
            pltpu.sync_copy(dstg_hbm.at[pl.ds(g0 + ci * CHUNK, CHUNK), :], dst_v)
            pltpu.async_copy(table_hbm.at[src_v.at[0]], rows0_v, sem0)
            pltpu.async_copy(table_hbm.at[src_v.at[1]], rows1_v, sem1)

            def pair(j2, _):
                e = 2 * j2
                pltpu.make_async_copy(table_hbm.at[src_v.at[e]],
                                      rows0_v, sem0).wait()
                pltpu.sync_copy(rows0_v, agg_sh.at[dst_v.at[e]], add=True)
                if with_cnt:
                    pltpu.sync_copy(ones_v, cnt_sh.at[dst_v.at[e]], add=True)

                @pl.when(e + 2 < CHUNK)
                def _start0():
                    pltpu.async_copy(table_hbm.at[src_v.at[e + 2]],
                                     rows0_v, sem0)

                pltpu.make_async_copy(table_hbm.at[src_v.at[e + 1]],
                                      rows1_v, sem1).wait()
                pltpu.sync_copy(rows1_v, agg_sh.at[dst_v.at[e + 1]], add=True)
                if with_cnt:
                    pltpu.sync_copy(ones_v, cnt_sh.at[dst_v.at[e + 1]], add=True)

                @pl.when(e + 3 < CHUNK)
                def _start1():
                    pltpu.async_copy(table_hbm.at[src_v.at[e + 3]],
                                     rows1_v, sem1)
                return _

            lax.fori_loop(0, CHUNK // 2, pair, None)
            return _

        lax.fori_loop(0, G_PER_TILE // CHUNK, do_chunk, None)
        plsc.subcore_barrier()

        pltpu.sync_copy(agg_sh.at[pl.ds(row0, ROWS_PER_TILE), :],
                        feat_out.at[c, pl.ds(row0, ROWS_PER_TILE), :])
        if with_cnt:
            pltpu.sync_copy(cnt_sh.at[pl.ds(row0, ROWS_PER_TILE)],
                            cnt_out.at[c, pl.ds(row0, ROWS_PER_TILE)])

    return pl.kernel(body, out_type=tuple(out_type), mesh=mesh,
                     scratch_types=tuple(scratch))


BLK = 5120


def _tc_mm(x, W, b):
    grid = (pl.cdiv(N_NODES, BLK),)

    def body(x_ref, w_ref, b_ref, o_ref):
        o_ref[...] = jnp.dot(x_ref[...], w_ref[...],
                             preferred_element_type=jnp.float32) + b_ref[...]

    return pl.pallas_call(
        body,
        grid=grid,
        in_specs=[
            pl.BlockSpec((BLK, D_FEAT), lambda i: (i, 0)),
            pl.BlockSpec((D_FEAT, D_FEAT), lambda i: (0, 0)),
            pl.BlockSpec((1, D_FEAT), lambda i: (0, 0)),
        ],
        out_specs=pl.BlockSpec((BLK, D_FEAT), lambda i: (i, 0)),
        out_shape=jax.ShapeDtypeStruct((N_NODES, D_FEAT), jnp.float32),
    )(x, W, b.reshape(1, D_FEAT))


def _tc_combine(feat_p, cnt_r, xr, Wl, relu: bool):
    grid = (pl.cdiv(N_NODES, BLK),)
    sub = BLK // LANES

    def body(feat_ref, cnt_ref, xr_ref, wl_ref, o_ref):
        i = pl.program_id(0)
        ssum = feat_ref[0] + feat_ref[1]
        ii = lax.broadcasted_iota(jnp.int32, (LANES, LANES), 0)
        jj = lax.broadcasted_iota(jnp.int32, (LANES, LANES), 1)
        eye = ii == jj
        parts = []
        for k in range(sub):
            cnt = cnt_ref[0, i * sub + k, :] + cnt_ref[1, i * sub + k, :]
            inv = (1.0 / jnp.maximum(cnt, 1.0))[None, :]
            dmat = jnp.where(eye, inv, 0.0)
            parts.append(jnp.dot(dmat, ssum[k * LANES:(k + 1) * LANES],
                                 preferred_element_type=jnp.float32))
        mean = jnp.concatenate(parts, axis=0)
        acc = jnp.dot(mean, wl_ref[...], preferred_element_type=jnp.float32)
        acc += xr_ref[...]
        o_ref[...] = jnp.maximum(acc, 0.0) if relu else acc

    return pl.pallas_call(
        body,
        grid=grid,
        in_specs=[
            pl.BlockSpec((NC, BLK, D_FEAT), lambda i: (0, i, 0)),
            pl.BlockSpec((NC, N_PAD // LANES, LANES), lambda i: (0, 0, 0)),
            pl.BlockSpec((BLK, D_FEAT), lambda i: (i, 0)),
            pl.BlockSpec((D_FEAT, D_FEAT), lambda i: (0, 0)),
        ],
        out_specs=pl.BlockSpec((BLK, D_FEAT), lambda i: (i, 0)),
        out_shape=jax.ShapeDtypeStruct((N_NODES, D_FEAT), jnp.float32),
    )(feat_p, cnt_r, xr, Wl)


def kernel(x, edge_index, W1l, b1, W1r, W2l, b2, W2r):
    src = edge_index[0].astype(jnp.int32)
    dst = edge_index[1].astype(jnp.int32)

    pad = E_PAD - N_EDGES
    ar = jnp.arange(pad, dtype=jnp.int32)
    src_p = jnp.concatenate([src, ar % N_NODES])
    dst_p = jnp.concatenate([dst, N_NODES + ar % (N_PAD - N_NODES)])
    srcg = src_p.reshape(G_TOTAL, LANES)
    dstg = dst_p.reshape(G_TOTAL, LANES)

    xr1 = _tc_mm(x, W1r, b1)
    feat1_p, cnt_p = _sc_segment_sum(True)(x, srcg, dstg)
    cnt_r = cnt_p.reshape(NC, N_PAD // LANES, LANES)
    h = _tc_combine(feat1_p, cnt_r, xr1, W1l, relu=True)
    xr2 = _tc_mm(h, W2r, b2)
    (feat2_p,) = _sc_segment_sum(False)(h, srcg, dstg)
    out = _tc_combine(feat2_p, cnt_r, xr2, W2l, relu=False)
    return out
